# baseline (speedup 1.0000x reference)
import functools

import jax
import jax.numpy as jnp
from jax import lax
from jax.experimental import pallas as pl
from jax.experimental.pallas import tpu as pltpu
from jax.experimental.pallas import tpu_sc as plsc

N_NODES = 10000
N_PAD = 10240
N_EDGES = 320000
D = 128
NUM_GRAPHS = 64
EPS = 1e-5

NC = 2
NS = 16
NW = NC * NS
EPT = N_EDGES // NW
CHUNK = 128
EPT_PAD = 10240
NCHUNK = EPT_PAD // CHUNK
DEG_CHUNK = 80
DEG_NCHUNK = EPT // DEG_CHUNK
ROWS_PT = N_PAD // NS
DEG_PT = N_PAD // NS


def _vsc_mesh():
  return plsc.VectorSubcoreMesh(core_axis_name="c", subcore_axis_name="s")


def _sc_deg(dst3d):
  @functools.partial(
      pl.kernel,
      mesh=_vsc_mesh(),
      out_type=jax.ShapeDtypeStruct((NC, N_PAD), jnp.float32),
      scratch_types=[
          pltpu.VMEM((DEG_NCHUNK, DEG_CHUNK), jnp.int32),
          pltpu.VMEM((DEG_CHUNK,), jnp.float32),
          pltpu.VMEM((DEG_PT,), jnp.float32),
          pltpu.VMEM_SHARED((N_PAD,), jnp.float32),
      ],
  )
  def deg_kernel(dst_hbm, out_hbm, dst_v, ones_v, buf_v, hist_s):
    cid = lax.axis_index("c")
    sid = lax.axis_index("s")
    tid = cid * NS + sid

    zeros16 = jnp.zeros((16,), jnp.float32)
    ones16 = jnp.ones((16,), jnp.float32)

    def fill(i, _):
      buf_v[pl.ds(i * 16, 16)] = zeros16
      return 0
    lax.fori_loop(0, DEG_PT // 16, fill, 0)
    for l in range(DEG_CHUNK // 16):
      ones_v[pl.ds(l * 16, 16)] = ones16

    base = sid * DEG_PT
    pltpu.sync_copy(buf_v, hist_s.at[pl.ds(base, DEG_PT)])
    pltpu.sync_copy(dst_hbm.at[tid], dst_v)
    plsc.subcore_barrier()

    def scat(j, _):
      pltpu.sync_copy(ones_v, hist_s.at[dst_v.at[j]], add=True)
      return 0
    lax.fori_loop(0, DEG_NCHUNK, scat, 0)

    plsc.subcore_barrier()
    pltpu.sync_copy(hist_s.at[pl.ds(base, DEG_PT)],
                    out_hbm.at[cid, pl.ds(base, DEG_PT)])

  return deg_kernel(dst3d)


def _sc_edge_pass(g, src3d, dst3d):
  @functools.partial(
      pl.kernel,
      mesh=_vsc_mesh(),
      out_type=jax.ShapeDtypeStruct((NC, N_PAD, D), jnp.float32),
      scratch_types=[
          pltpu.VMEM((NCHUNK, CHUNK), jnp.int32),
          pltpu.VMEM((NCHUNK, CHUNK), jnp.int32),
          pltpu.VMEM((CHUNK, D), jnp.float32),
          pltpu.VMEM_SHARED((N_PAD, D), jnp.float32),
      ],
  )
  def edge_kernel(g_hbm, src_hbm, dst_hbm, out_hbm, src_v, dst_v, rows_v,
                  acc_s):
    cid = lax.axis_index("c")
    sid = lax.axis_index("s")
    tid = cid * NS + sid

    zeros16 = jnp.zeros((16,), jnp.float32)

    def zrow(i, _):
      for l in range(D // 16):
        rows_v[i, pl.ds(l * 16, 16)] = zeros16
      return 0
    lax.fori_loop(0, CHUNK, zrow, 0)
    def zcopy(i, _):
      pltpu.sync_copy(rows_v, acc_s.at[pl.ds(sid * ROWS_PT + i * CHUNK, CHUNK)])
      return 0
    lax.fori_loop(0, ROWS_PT // CHUNK, zcopy, 0)

    pltpu.sync_copy(src_hbm.at[tid], src_v)
    pltpu.sync_copy(dst_hbm.at[tid], dst_v)
    plsc.subcore_barrier()

    def step(c, _):
      pltpu.sync_copy(g_hbm.at[src_v.at[c]], rows_v)
      pltpu.sync_copy(rows_v, acc_s.at[dst_v.at[c]], add=True)
      return 0
    lax.fori_loop(0, NCHUNK, step, 0)

    plsc.subcore_barrier()
    pltpu.sync_copy(acc_s.at[pl.ds(sid * ROWS_PT, ROWS_PT)],
                    out_hbm.at[cid, pl.ds(sid * ROWS_PT, ROWS_PT)])

  return edge_kernel(g, src3d, dst3d)


def _dinv_from(deg2):
  deg = deg2[0, :N_NODES] + deg2[1, :N_NODES] + 1.0
  return lax.rsqrt(deg)


def _bn(h, gamma, beta):
  m = jnp.mean(h, axis=0)
  v = jnp.mean((h - m) ** 2, axis=0)
  return (h - m) / jnp.sqrt(v + EPS) * gamma + beta


def _tc_pre(deg2, x, W1):
  def body(deg_ref, x_ref, w_ref, g_ref):
    dinv = _dinv_from(deg_ref[...])
    hw = jnp.dot(x_ref[...], w_ref[...], preferred_element_type=jnp.float32)
    g_ref[...] = hw * dinv[:, None]
  return pl.pallas_call(
      body, out_shape=jax.ShapeDtypeStruct((N_NODES, D), jnp.float32),
  )(deg2, x, W1)


def _tc_mid(acc2, g1, deg2, b1, gm1, bt1, W2):
  def body(acc_ref, g_ref, deg_ref, b_ref, gm_ref, bt_ref, w_ref, out_ref):
    dinv = _dinv_from(deg_ref[...])
    acc = acc_ref[0, :N_NODES] + acc_ref[1, :N_NODES]
    pre = dinv[:, None] * (acc + g_ref[...]) + b_ref[...]
    h1 = jax.nn.relu(_bn(pre, gm_ref[...], bt_ref[...]))
    hw = jnp.dot(h1, w_ref[...], preferred_element_type=jnp.float32)
    out_ref[...] = hw * dinv[:, None]
  return pl.pallas_call(
      body, out_shape=jax.ShapeDtypeStruct((N_NODES, D), jnp.float32),
  )(acc2, g1, deg2, b1, gm1, bt1, W2)


def _tc_post(acc2, g2, deg2, b2, gm2, bt2, batch2d,
             LW1, Lb1, Lg1, Lbe1, LW2, Lb2, Lg2, Lbe2):
  def body(acc_ref, g_ref, deg_ref, b_ref, gm_ref, bt_ref, batch_ref,
           lw1_ref, lb1_ref, lg1_ref, lbe1_ref, lw2_ref, lb2_ref, lg2_ref,
           lbe2_ref, out_ref):
    dinv = _dinv_from(deg_ref[...])
    acc = acc_ref[0, :N_NODES] + acc_ref[1, :N_NODES]
    pre = dinv[:, None] * (acc + g_ref[...]) + b_ref[...]
    h2 = jax.nn.relu(_bn(pre, gm_ref[...], bt_ref[...]))
    gids = lax.broadcasted_iota(jnp.int32, (NUM_GRAPHS, N_NODES), 0)
    P = (gids == batch_ref[...]).astype(jnp.float32)
    cnts = jnp.sum(P, axis=1)
    sums = jnp.dot(P, h2, preferred_element_type=jnp.float32, precision=lax.Precision.HIGHEST)
    pooled = sums / jnp.maximum(cnts, 1.0)[:, None]
    z = jnp.dot(pooled, lw1_ref[...], preferred_element_type=jnp.float32)
    z = jax.nn.relu(_bn(z + lb1_ref[...], lg1_ref[...], lbe1_ref[...]))
    z = jnp.dot(z, lw2_ref[...], preferred_element_type=jnp.float32)
    out_ref[...] = _bn(z + lb2_ref[...], lg2_ref[...], lbe2_ref[...])
  return pl.pallas_call(
      body, out_shape=jax.ShapeDtypeStruct((NUM_GRAPHS, 1), jnp.float32),
  )(acc2, g2, deg2, b2, gm2, bt2, batch2d,
    LW1, Lb1, Lg1, Lbe1, LW2, Lb2, Lg2, Lbe2)


def kernel(x, edge_index, batch, W1, b1, g1, be1, W2, b2, g2, be2,
           LW1, Lb1, Lg1, Lbe1, LW2, Lb2, Lg2, Lbe2):
  src = edge_index[0].astype(jnp.int32)
  dst = edge_index[1].astype(jnp.int32)
  npad_e = NW * EPT_PAD - N_EDGES
  src_pad = jnp.concatenate(
      [src.reshape(NW, EPT),
       jnp.zeros((NW, npad_e // NW), jnp.int32)], axis=1)
  trash = N_NODES + jnp.arange(NW, dtype=jnp.int32)[:, None]
  dst_pad = jnp.concatenate(
      [dst.reshape(NW, EPT),
       jnp.broadcast_to(trash, (NW, npad_e // NW))], axis=1)
  src3d = src_pad.reshape(NW, NCHUNK, CHUNK)
  dst3d = dst_pad.reshape(NW, NCHUNK, CHUNK)
  dst3d_deg = dst.reshape(NW, DEG_NCHUNK, DEG_CHUNK)
  batch2d = batch.astype(jnp.int32).reshape(1, N_NODES)

  deg2 = _sc_deg(dst3d_deg)
  gm1 = _tc_pre(deg2, x, W1)
  acc1 = _sc_edge_pass(gm1, src3d, dst3d)
  gm2 = _tc_mid(acc1, gm1, deg2, b1, g1, be1, W2)
  acc2 = _sc_edge_pass(gm2, src3d, dst3d)
  return _tc_post(acc2, gm2, deg2, b2, g2, be2, batch2d,
                  LW1, Lb1, Lg1, Lbe1, LW2, Lb2, Lg2, Lbe2)


# speedup vs baseline: 2.0283x; 2.0283x over previous
import functools

import jax
import jax.numpy as jnp
from jax import lax
from jax.experimental import pallas as pl
from jax.experimental.pallas import tpu as pltpu
from jax.experimental.pallas import tpu_sc as plsc

N_NODES = 10000
N_PAD = 10240
N_EDGES = 320000
D = 128
NUM_GRAPHS = 64
EPS = 1e-5

NC = 2
NS = 16
NW = NC * NS
EPT = N_EDGES // NW
CHUNK = 80
NCHUNK = EPT // CHUNK
DEG_CHUNK = 80
DEG_NCHUNK = EPT // DEG_CHUNK
ROWS_PT = N_PAD // NS
DEG_PT = N_PAD // NS


def _vsc_mesh():
  return plsc.VectorSubcoreMesh(core_axis_name="c", subcore_axis_name="s")


def _sc_deg(dst3d):
  @functools.partial(
      pl.kernel,
      mesh=_vsc_mesh(),
      out_type=jax.ShapeDtypeStruct((NC, N_PAD), jnp.float32),
      scratch_types=[
          pltpu.VMEM((DEG_NCHUNK, DEG_CHUNK), jnp.int32),
          pltpu.VMEM((DEG_CHUNK,), jnp.float32),
          pltpu.VMEM((DEG_PT,), jnp.float32),
          pltpu.VMEM_SHARED((N_PAD,), jnp.float32),
      ],
  )
  def deg_kernel(dst_hbm, out_hbm, dst_v, ones_v, buf_v, hist_s):
    cid = lax.axis_index("c")
    sid = lax.axis_index("s")
    tid = cid * NS + sid

    zeros16 = jnp.zeros((16,), jnp.float32)
    ones16 = jnp.ones((16,), jnp.float32)

    def fill(i, _):
      buf_v[pl.ds(i * 16, 16)] = zeros16
      return 0
    lax.fori_loop(0, DEG_PT // 16, fill, 0)
    for l in range(DEG_CHUNK // 16):
      ones_v[pl.ds(l * 16, 16)] = ones16

    base = sid * DEG_PT
    pltpu.sync_copy(buf_v, hist_s.at[pl.ds(base, DEG_PT)])
    pltpu.sync_copy(dst_hbm.at[tid], dst_v)
    plsc.subcore_barrier()

    def scat(j, _):
      pltpu.sync_copy(ones_v, hist_s.at[dst_v.at[j]], add=True)
      return 0
    lax.fori_loop(0, DEG_NCHUNK, scat, 0)

    plsc.subcore_barrier()
    pltpu.sync_copy(hist_s.at[pl.ds(base, DEG_PT)],
                    out_hbm.at[cid, pl.ds(base, DEG_PT)])

  return deg_kernel(dst3d)


def _sc_edge_pass(g, src3d, dst3d):
  @functools.partial(
      pl.kernel,
      mesh=_vsc_mesh(),
      out_type=jax.ShapeDtypeStruct((NC, N_PAD, D), jnp.float32),
      scratch_types=[
          pltpu.VMEM((NCHUNK, CHUNK), jnp.int32),
          pltpu.VMEM((NCHUNK, CHUNK), jnp.int32),
          pltpu.VMEM((CHUNK, D), jnp.float32),
          pltpu.VMEM_SHARED((N_PAD, D), jnp.float32),
      ],
  )
  def edge_kernel(g_hbm, src_hbm, dst_hbm, out_hbm, src_v, dst_v, rows_v,
                  acc_s):
    cid = lax.axis_index("c")
    sid = lax.axis_index("s")
    tid = cid * NS + sid

    zeros16 = jnp.zeros((16,), jnp.float32)

    def zrow(i, _):
      for l in range(D // 16):
        rows_v[i, pl.ds(l * 16, 16)] = zeros16
      return 0
    lax.fori_loop(0, CHUNK, zrow, 0)
    def zcopy(i, _):
      pltpu.sync_copy(rows_v, acc_s.at[pl.ds(sid * ROWS_PT + i * CHUNK, CHUNK)])
      return 0
    lax.fori_loop(0, ROWS_PT // CHUNK, zcopy, 0)

    pltpu.sync_copy(src_hbm.at[tid], src_v)
    pltpu.sync_copy(dst_hbm.at[tid], dst_v)
    plsc.subcore_barrier()

    def step(c, _):
      pltpu.sync_copy(g_hbm.at[src_v.at[c]], rows_v)
      pltpu.sync_copy(rows_v, acc_s.at[dst_v.at[c]], add=True)
      return 0
    lax.fori_loop(0, NCHUNK, step, 0)

    plsc.subcore_barrier()
    pltpu.sync_copy(acc_s.at[pl.ds(sid * ROWS_PT, ROWS_PT)],
                    out_hbm.at[cid, pl.ds(sid * ROWS_PT, ROWS_PT)])

  return edge_kernel(g, src3d, dst3d)


def _dinv_from(deg2):
  deg = deg2[0, :N_NODES] + deg2[1, :N_NODES] + 1.0
  return lax.rsqrt(deg)


def _bn(h, gamma, beta):
  m = jnp.mean(h, axis=0)
  v = jnp.mean((h - m) ** 2, axis=0)
  return (h - m) / jnp.sqrt(v + EPS) * gamma + beta


def _tc_pre(deg2, x, W1):
  def body(deg_ref, x_ref, w_ref, g_ref):
    dinv = _dinv_from(deg_ref[...])
    hw = jnp.dot(x_ref[...], w_ref[...], preferred_element_type=jnp.float32)
    g_ref[...] = hw * dinv[:, None]
  return pl.pallas_call(
      body, out_shape=jax.ShapeDtypeStruct((N_NODES, D), jnp.float32),
  )(deg2, x, W1)


def _tc_mid(acc2, g1, deg2, b1, gm1, bt1, W2):
  def body(acc_ref, g_ref, deg_ref, b_ref, gm_ref, bt_ref, w_ref, out_ref):
    dinv = _dinv_from(deg_ref[...])
    acc = acc_ref[0, :N_NODES] + acc_ref[1, :N_NODES]
    pre = dinv[:, None] * (acc + g_ref[...]) + b_ref[...]
    h1 = jax.nn.relu(_bn(pre, gm_ref[...], bt_ref[...]))
    hw = jnp.dot(h1, w_ref[...], preferred_element_type=jnp.float32)
    out_ref[...] = hw * dinv[:, None]
  return pl.pallas_call(
      body, out_shape=jax.ShapeDtypeStruct((N_NODES, D), jnp.float32),
  )(acc2, g1, deg2, b1, gm1, bt1, W2)


def _tc_post(acc2, g2, deg2, b2, gm2, bt2, batch2d,
             LW1, Lb1, Lg1, Lbe1, LW2, Lb2, Lg2, Lbe2):
  def body(acc_ref, g_ref, deg_ref, b_ref, gm_ref, bt_ref, batch_ref,
           lw1_ref, lb1_ref, lg1_ref, lbe1_ref, lw2_ref, lb2_ref, lg2_ref,
           lbe2_ref, out_ref):
    dinv = _dinv_from(deg_ref[...])
    acc = acc_ref[0, :N_NODES] + acc_ref[1, :N_NODES]
    pre = dinv[:, None] * (acc + g_ref[...]) + b_ref[...]
    h2 = jax.nn.relu(_bn(pre, gm_ref[...], bt_ref[...]))
    gids = lax.broadcasted_iota(jnp.int32, (NUM_GRAPHS, N_NODES), 0)
    P = (gids == batch_ref[...]).astype(jnp.float32)
    cnts = jnp.sum(P, axis=1)
    sums = jnp.dot(P, h2, preferred_element_type=jnp.float32, precision=lax.Precision.HIGHEST)
    pooled = sums / jnp.maximum(cnts, 1.0)[:, None]
    z = jnp.dot(pooled, lw1_ref[...], preferred_element_type=jnp.float32)
    z = jax.nn.relu(_bn(z + lb1_ref[...], lg1_ref[...], lbe1_ref[...]))
    z = jnp.dot(z, lw2_ref[...], preferred_element_type=jnp.float32)
    out_ref[...] = _bn(z + lb2_ref[...], lg2_ref[...], lbe2_ref[...])
  return pl.pallas_call(
      body, out_shape=jax.ShapeDtypeStruct((NUM_GRAPHS, 1), jnp.float32),
  )(acc2, g2, deg2, b2, gm2, bt2, batch2d,
    LW1, Lb1, Lg1, Lbe1, LW2, Lb2, Lg2, Lbe2)


def kernel(x, edge_index, batch, W1, b1, g1, be1, W2, b2, g2, be2,
           LW1, Lb1, Lg1, Lbe1, LW2, Lb2, Lg2, Lbe2):
  src3d = edge_index[0].astype(jnp.int32).reshape(NW, NCHUNK, CHUNK)
  dst3d = edge_index[1].astype(jnp.int32).reshape(NW, NCHUNK, CHUNK)
  dst3d_deg = edge_index[1].astype(jnp.int32).reshape(NW, DEG_NCHUNK, DEG_CHUNK)
  batch2d = batch.astype(jnp.int32).reshape(1, N_NODES)

  deg2 = _sc_deg(dst3d_deg)
  gm1 = _tc_pre(deg2, x, W1)
  acc1 = _sc_edge_pass(gm1, src3d, dst3d)
  gm2 = _tc_mid(acc1, gm1, deg2, b1, g1, be1, W2)
  acc2 = _sc_edge_pass(gm2, src3d, dst3d)
  return _tc_post(acc2, gm2, deg2, b2, g2, be2, batch2d,
                  LW1, Lb1, Lg1, Lbe1, LW2, Lb2, Lg2, Lbe2)


# speedup vs baseline: 2.5620x; 1.2631x over previous
import functools

import jax
import jax.numpy as jnp
from jax import lax
from jax.experimental import pallas as pl
from jax.experimental.pallas import tpu as pltpu
from jax.experimental.pallas import tpu_sc as plsc

N_NODES = 10000
N_PAD = 10240
N_EDGES = 320000
D = 128
NUM_GRAPHS = 64
EPS = 1e-5

NC = 2
NS = 16
NW = NC * NS
EPT = N_EDGES // NW
CHUNK = 80
NCHUNK = EPT // CHUNK
DEG_CHUNK = 80
DEG_NCHUNK = EPT // DEG_CHUNK
ROWS_PT = N_PAD // NS
DEG_PT = N_PAD // NS


def _vsc_mesh():
  return plsc.VectorSubcoreMesh(core_axis_name="c", subcore_axis_name="s")


def _sc_deg(dst3d):
  @functools.partial(
      pl.kernel,
      mesh=_vsc_mesh(),
      out_type=jax.ShapeDtypeStruct((NC, N_PAD), jnp.float32),
      scratch_types=[
          pltpu.VMEM((DEG_NCHUNK, DEG_CHUNK), jnp.int32),
          pltpu.VMEM((DEG_CHUNK,), jnp.float32),
          pltpu.VMEM((DEG_PT,), jnp.float32),
          pltpu.VMEM_SHARED((N_PAD,), jnp.float32),
      ],
  )
  def deg_kernel(dst_hbm, out_hbm, dst_v, ones_v, buf_v, hist_s):
    cid = lax.axis_index("c")
    sid = lax.axis_index("s")
    tid = cid * NS + sid

    zeros16 = jnp.zeros((16,), jnp.float32)
    ones16 = jnp.ones((16,), jnp.float32)

    def fill(i, _):
      buf_v[pl.ds(i * 16, 16)] = zeros16
      return 0
    lax.fori_loop(0, DEG_PT // 16, fill, 0)
    for l in range(DEG_CHUNK // 16):
      ones_v[pl.ds(l * 16, 16)] = ones16

    base = sid * DEG_PT
    pltpu.sync_copy(buf_v, hist_s.at[pl.ds(base, DEG_PT)])
    pltpu.sync_copy(dst_hbm.at[tid], dst_v)
    plsc.subcore_barrier()

    def scat(j, _):
      pltpu.sync_copy(ones_v, hist_s.at[dst_v.at[j]], add=True)
      return 0
    lax.fori_loop(0, DEG_NCHUNK, scat, 0)

    plsc.subcore_barrier()
    pltpu.sync_copy(hist_s.at[pl.ds(base, DEG_PT)],
                    out_hbm.at[cid, pl.ds(base, DEG_PT)])

  return deg_kernel(dst3d)


def _sc_edge_pass(g, src3d, dst3d):
  @functools.partial(
      pl.kernel,
      mesh=_vsc_mesh(),
      out_type=jax.ShapeDtypeStruct((NC, N_PAD, D), jnp.float32),
      scratch_types=[
          pltpu.VMEM((EPT,), jnp.int32),
          pltpu.VMEM((NCHUNK, CHUNK), jnp.int32),
          pltpu.VMEM((2, CHUNK, D), jnp.float32),
          pltpu.VMEM_SHARED((N_PAD, D), jnp.float32),
          pltpu.SemaphoreType.DMA,
      ],
  )
  def edge_kernel(g_hbm, src_hbm, dst_hbm, out_hbm, src_v, dst_v, rows_v,
                  acc_s, gsem):
    cid = lax.axis_index("c")
    sid = lax.axis_index("s")
    tid = cid * NS + sid

    zeros16 = jnp.zeros((16,), jnp.float32)

    def zrow(i, _):
      for l in range(D // 16):
        rows_v[0, i, pl.ds(l * 16, 16)] = zeros16
      return 0
    lax.fori_loop(0, CHUNK, zrow, 0)
    def zcopy(i, _):
      pltpu.sync_copy(rows_v.at[0],
                      acc_s.at[pl.ds(sid * ROWS_PT + i * CHUNK, CHUNK)])
      return 0
    lax.fori_loop(0, ROWS_PT // CHUNK, zcopy, 0)

    pltpu.sync_copy(src_hbm.at[tid], src_v)
    pltpu.sync_copy(dst_hbm.at[tid], dst_v)
    plsc.subcore_barrier()

    pltpu.async_copy(g_hbm.at[src_v.at[pl.ds(0, CHUNK)]], rows_v.at[0], gsem)

    def step(c, _):
      b = lax.rem(c, 2)
      nb = 1 - b
      pltpu.make_async_copy(g_hbm.at[src_v.at[pl.ds(c * CHUNK, CHUNK)]],
                            rows_v.at[b], gsem).wait()
      @pl.when(c < NCHUNK - 1)
      def _():
        pltpu.async_copy(g_hbm.at[src_v.at[pl.ds((c + 1) * CHUNK, CHUNK)]],
                         rows_v.at[nb], gsem)
      pltpu.sync_copy(rows_v.at[b], acc_s.at[dst_v.at[c]], add=True)
      return 0
    lax.fori_loop(0, NCHUNK, step, 0)

    plsc.subcore_barrier()
    pltpu.sync_copy(acc_s.at[pl.ds(sid * ROWS_PT, ROWS_PT)],
                    out_hbm.at[cid, pl.ds(sid * ROWS_PT, ROWS_PT)])

  return edge_kernel(g, src3d, dst3d)


def _dinv_from(deg2):
  deg = deg2[0, :N_NODES] + deg2[1, :N_NODES] + 1.0
  return lax.rsqrt(deg)


def _bn(h, gamma, beta):
  m = jnp.mean(h, axis=0)
  v = jnp.mean((h - m) ** 2, axis=0)
  return (h - m) / jnp.sqrt(v + EPS) * gamma + beta


def _tc_pre(deg2, x, W1):
  def body(deg_ref, x_ref, w_ref, g_ref):
    dinv = _dinv_from(deg_ref[...])
    hw = jnp.dot(x_ref[...], w_ref[...], preferred_element_type=jnp.float32)
    g_ref[...] = hw * dinv[:, None]
  return pl.pallas_call(
      body, out_shape=jax.ShapeDtypeStruct((N_NODES, D), jnp.float32),
  )(deg2, x, W1)


def _tc_mid(acc2, g1, deg2, b1, gm1, bt1, W2):
  def body(acc_ref, g_ref, deg_ref, b_ref, gm_ref, bt_ref, w_ref, out_ref):
    dinv = _dinv_from(deg_ref[...])
    acc = acc_ref[0, :N_NODES] + acc_ref[1, :N_NODES]
    pre = dinv[:, None] * (acc + g_ref[...]) + b_ref[...]
    h1 = jax.nn.relu(_bn(pre, gm_ref[...], bt_ref[...]))
    hw = jnp.dot(h1, w_ref[...], preferred_element_type=jnp.float32)
    out_ref[...] = hw * dinv[:, None]
  return pl.pallas_call(
      body, out_shape=jax.ShapeDtypeStruct((N_NODES, D), jnp.float32),
  )(acc2, g1, deg2, b1, gm1, bt1, W2)


def _tc_post(acc2, g2, deg2, b2, gm2, bt2, batch2d,
             LW1, Lb1, Lg1, Lbe1, LW2, Lb2, Lg2, Lbe2):
  def body(acc_ref, g_ref, deg_ref, b_ref, gm_ref, bt_ref, batch_ref,
           lw1_ref, lb1_ref, lg1_ref, lbe1_ref, lw2_ref, lb2_ref, lg2_ref,
           lbe2_ref, out_ref):
    dinv = _dinv_from(deg_ref[...])
    acc = acc_ref[0, :N_NODES] + acc_ref[1, :N_NODES]
    pre = dinv[:, None] * (acc + g_ref[...]) + b_ref[...]
    h2 = jax.nn.relu(_bn(pre, gm_ref[...], bt_ref[...]))
    gids = lax.broadcasted_iota(jnp.int32, (NUM_GRAPHS, N_NODES), 0)
    P = (gids == batch_ref[...]).astype(jnp.float32)
    cnts = jnp.sum(P, axis=1)
    sums = jnp.dot(P, h2, preferred_element_type=jnp.float32, precision=lax.Precision.HIGHEST)
    pooled = sums / jnp.maximum(cnts, 1.0)[:, None]
    z = jnp.dot(pooled, lw1_ref[...], preferred_element_type=jnp.float32)
    z = jax.nn.relu(_bn(z + lb1_ref[...], lg1_ref[...], lbe1_ref[...]))
    z = jnp.dot(z, lw2_ref[...], preferred_element_type=jnp.float32)
    out_ref[...] = _bn(z + lb2_ref[...], lg2_ref[...], lbe2_ref[...])
  return pl.pallas_call(
      body, out_shape=jax.ShapeDtypeStruct((NUM_GRAPHS, 1), jnp.float32),
  )(acc2, g2, deg2, b2, gm2, bt2, batch2d,
    LW1, Lb1, Lg1, Lbe1, LW2, Lb2, Lg2, Lbe2)


def kernel(x, edge_index, batch, W1, b1, g1, be1, W2, b2, g2, be2,
           LW1, Lb1, Lg1, Lbe1, LW2, Lb2, Lg2, Lbe2):
  src3d = edge_index[0].astype(jnp.int32).reshape(NW, EPT)
  dst3d = edge_index[1].astype(jnp.int32).reshape(NW, NCHUNK, CHUNK)
  dst3d_deg = edge_index[1].astype(jnp.int32).reshape(NW, DEG_NCHUNK, DEG_CHUNK)
  batch2d = batch.astype(jnp.int32).reshape(1, N_NODES)

  deg2 = _sc_deg(dst3d_deg)
  gm1 = _tc_pre(deg2, x, W1)
  acc1 = _sc_edge_pass(gm1, src3d, dst3d)
  gm2 = _tc_mid(acc1, gm1, deg2, b1, g1, be1, W2)
  acc2 = _sc_edge_pass(gm2, src3d, dst3d)
  return _tc_post(acc2, gm2, deg2, b2, g2, be2, batch2d,
                  LW1, Lb1, Lg1, Lbe1, LW2, Lb2, Lg2, Lbe2)
